# baseline (speedup 1.0000x reference)
import functools

import jax
import jax.numpy as jnp
from jax.experimental import pallas as pl
from jax.experimental.pallas import tpu as pltpu
from jax.experimental.pallas import tpu_sc as plsc

_mesh = plsc.VectorSubcoreMesh(
    core_axis_name="c", subcore_axis_name="s", num_cores=1, num_subcores=1
)


@functools.partial(
    pl.kernel,
    mesh=_mesh,
    out_type=jax.ShapeDtypeStruct((4, 2, 2, 3), jnp.float32),
    scratch_types=[pltpu.SemaphoreType.DMA],
)
def _scatter_overwrite(x_hbm, z_hbm, out_hbm, sem):
    copies = [
        pltpu.async_copy(z_hbm, out_hbm.at[0], sem),
        pltpu.async_copy(x_hbm, out_hbm.at[1], sem),
        pltpu.async_copy(x_hbm, out_hbm.at[2], sem),
        pltpu.async_copy(z_hbm, out_hbm.at[3], sem),
    ]
    for c in copies:
        c.wait()


def kernel(x):
    return _scatter_overwrite(x, jnp.zeros((2, 2, 3), jnp.float32))


# speedup vs baseline: 1.0129x; 1.0129x over previous
import functools

import jax
import jax.numpy as jnp
import numpy as np
from jax.experimental import pallas as pl
from jax.experimental.pallas import tpu as pltpu
from jax.experimental.pallas import tpu_sc as plsc

_mesh = plsc.VectorSubcoreMesh(
    core_axis_name="c", subcore_axis_name="s", num_cores=1, num_subcores=1
)


@functools.partial(
    pl.kernel,
    mesh=_mesh,
    out_type=jax.ShapeDtypeStruct((4, 2, 2, 3), jnp.float32),
    scratch_types=[pltpu.SemaphoreType.DMA],
    compiler_params=pltpu.CompilerParams(use_tc_tiling_on_sc=True),
)
def _scatter_overwrite(x_hbm, z_hbm, out_hbm, sem):
    copies = [
        pltpu.async_copy(z_hbm, out_hbm.at[0], sem),
        pltpu.async_copy(x_hbm, out_hbm.at[1], sem),
        pltpu.async_copy(x_hbm, out_hbm.at[2], sem),
        pltpu.async_copy(z_hbm, out_hbm.at[3], sem),
    ]
    for c in copies:
        c.wait()


_ZERO_ROW = np.zeros((2, 2, 3), np.float32)


def kernel(x):
    return _scatter_overwrite(x, _ZERO_ROW)


# speedup vs baseline: 1.0888x; 1.0749x over previous
import functools

import jax
import jax.numpy as jnp
import numpy as np
from jax.experimental import pallas as pl
from jax.experimental.pallas import tpu as pltpu
from jax.experimental.pallas import tpu_sc as plsc

_mesh = plsc.ScalarSubcoreMesh(axis_name="c", num_cores=1)


@functools.partial(
    pl.kernel,
    mesh=_mesh,
    out_type=jax.ShapeDtypeStruct((4, 2, 2, 3), jnp.float32),
    scratch_types=[pltpu.SemaphoreType.DMA],
    compiler_params=pltpu.CompilerParams(use_tc_tiling_on_sc=True),
)
def _scatter_overwrite(x_hbm, z_hbm, out_hbm, sem):
    copies = [
        pltpu.async_copy(z_hbm, out_hbm.at[0], sem),
        pltpu.async_copy(x_hbm, out_hbm.at[1], sem),
        pltpu.async_copy(x_hbm, out_hbm.at[2], sem),
        pltpu.async_copy(z_hbm, out_hbm.at[3], sem),
    ]
    for c in copies:
        c.wait()


_ZERO_ROW = np.zeros((2, 2, 3), np.float32)


def kernel(x):
    return _scatter_overwrite(x, _ZERO_ROW)


# speedup vs baseline: 1.1150x; 1.0240x over previous
import functools

import jax
import jax.numpy as jnp
import numpy as np
from jax.experimental import pallas as pl
from jax.experimental.pallas import tpu as pltpu
from jax.experimental.pallas import tpu_sc as plsc

_mesh = plsc.ScalarSubcoreMesh(axis_name="c", num_cores=1)


@functools.partial(
    pl.kernel,
    mesh=_mesh,
    out_type=jax.ShapeDtypeStruct((4, 2, 2, 3), jnp.float32),
    scratch_types=[pltpu.SemaphoreType.DMA],
    compiler_params=pltpu.CompilerParams(use_tc_tiling_on_sc=False),
)
def _scatter_overwrite(x_hbm, z_hbm, out_hbm, sem):
    copies = [
        pltpu.async_copy(z_hbm, out_hbm.at[0], sem),
        pltpu.async_copy(x_hbm, out_hbm.at[1], sem),
        pltpu.async_copy(x_hbm, out_hbm.at[2], sem),
        pltpu.async_copy(z_hbm, out_hbm.at[3], sem),
    ]
    for c in copies:
        c.wait()


_ZERO_ROW = np.zeros((2, 2, 3), np.float32)


def kernel(x):
    return _scatter_overwrite(x, _ZERO_ROW)


# speedup vs baseline: 1.1208x; 1.0052x over previous
import functools

import jax
import jax.numpy as jnp
import numpy as np
from jax.experimental import pallas as pl
from jax.experimental.pallas import tpu as pltpu
from jax.experimental.pallas import tpu_sc as plsc

_mesh = plsc.ScalarSubcoreMesh(axis_name="c", num_cores=1)


@functools.partial(
    pl.kernel,
    mesh=_mesh,
    out_type=jax.ShapeDtypeStruct((4, 2, 2, 3), jnp.float32),
    scratch_types=[pltpu.SemaphoreType.DMA],
    compiler_params=pltpu.CompilerParams(use_tc_tiling_on_sc=False, skip_device_barrier=True),
)
def _scatter_overwrite(x_hbm, z_hbm, out_hbm, sem):
    copies = [
        pltpu.async_copy(z_hbm, out_hbm.at[0], sem),
        pltpu.async_copy(x_hbm, out_hbm.at[1], sem),
        pltpu.async_copy(x_hbm, out_hbm.at[2], sem),
        pltpu.async_copy(z_hbm, out_hbm.at[3], sem),
    ]
    for c in copies:
        c.wait()


_ZERO_ROW = np.zeros((2, 2, 3), np.float32)


def kernel(x):
    return _scatter_overwrite(x, _ZERO_ROW)


# speedup vs baseline: 1.1375x; 1.0149x over previous
import functools

import jax
import jax.numpy as jnp
from jax.experimental import pallas as pl
from jax.experimental.pallas import tpu as pltpu
from jax.experimental.pallas import tpu_sc as plsc

_mesh = plsc.ScalarSubcoreMesh(axis_name="c", num_cores=1)


@functools.partial(
    pl.kernel,
    mesh=_mesh,
    out_type=jax.ShapeDtypeStruct((4, 2, 2, 3), jnp.float32),
    scratch_types=[
        pltpu.SMEM((2, 2, 3), jnp.float32),
        pltpu.SemaphoreType.DMA,
    ],
)
def _scatter_overwrite(x_hbm, out_hbm, z_smem, sem):
    for i in range(2):
        for j in range(2):
            for k in range(3):
                z_smem[i, j, k] = 0.0
    copies = [
        pltpu.async_copy(z_smem, out_hbm.at[0], sem),
        pltpu.async_copy(x_hbm, out_hbm.at[1], sem),
        pltpu.async_copy(x_hbm, out_hbm.at[2], sem),
        pltpu.async_copy(z_smem, out_hbm.at[3], sem),
    ]
    for c in copies:
        c.wait()


def kernel(x):
    return _scatter_overwrite(x)
